# Initial kernel scaffold; baseline (speedup 1.0000x reference)
import jax
import jax.numpy as jnp
import numpy as np
from jax.experimental import pallas as pl
from jax.experimental.pallas import tpu as pltpu

_N_POINTS = 1000000
_N_VOXELS = 120000
_OUT_CH = 64
_LO = np.array([-51.2, -51.2, -4.0], dtype=np.float32)
_INTERVALS = np.array([102.4 / 512.0, 102.4 / 512.0, 6.4 / 32.0], dtype=np.float32)

_TILE = 2000


def _mlp_body(pts_ref, gc_ref, mean_ref, w1_ref, b1_ref, w2_ref, b2_ref, out_ref):
    pts = pts_ref[...]
    xyz = pts[:, :3]
    centers = gc_ref[...] * _INTERVALS[None, :] + _LO[None, :]
    nor = xyz - mean_ref[...]
    ctp = xyz - centers
    zeros = jnp.zeros((pts.shape[0], 6), dtype=jnp.float32)
    pcf = jnp.concatenate([pts, nor, ctp, zeros], axis=1)
    h = jnp.maximum(
        jnp.dot(pcf, w1_ref[...], preferred_element_type=jnp.float32) + b1_ref[...],
        0.0,
    )
    out_ref[...] = (
        jnp.dot(h, w2_ref[...], preferred_element_type=jnp.float32) + b2_ref[...]
    )


def _run_mlp(points, gridf, pc_mean, W1p, b1, W2, b2):
    grid = _N_POINTS // _TILE
    return pl.pallas_call(
        _mlp_body,
        grid=(grid,),
        in_specs=[
            pl.BlockSpec((_TILE, 4), lambda i: (i, 0)),
            pl.BlockSpec((_TILE, 3), lambda i: (i, 0)),
            pl.BlockSpec((_TILE, 3), lambda i: (i, 0)),
            pl.BlockSpec((16, _OUT_CH), lambda i: (0, 0)),
            pl.BlockSpec((1, _OUT_CH), lambda i: (0, 0)),
            pl.BlockSpec((_OUT_CH, _OUT_CH), lambda i: (0, 0)),
            pl.BlockSpec((1, _OUT_CH), lambda i: (0, 0)),
        ],
        out_specs=pl.BlockSpec((_TILE, _OUT_CH), lambda i: (i, 0)),
        out_shape=jax.ShapeDtypeStruct((_N_POINTS, _OUT_CH), jnp.float32),
    )(points, gridf, pc_mean, W1p, b1, W2, b2)


def kernel(points, full_coors, coors_inv, coors, unmask_index, batch_size, W1, b1, W2, b2):
    xyz = points[:, :3]
    ones = jnp.ones((_N_POINTS, 1), dtype=jnp.float32)
    acc = jax.ops.segment_sum(
        jnp.concatenate([xyz, ones], axis=1), coors_inv, num_segments=_N_VOXELS
    )
    vox_mean = acc[:, :3] / jnp.clip(acc[:, 3:4], 1.0)
    pc_mean = vox_mean[coors_inv]

    gridf = full_coors[:, 1:].astype(jnp.float32)
    W1p = jnp.pad(W1, ((0, 6), (0, 0)))
    pt_fea = _run_mlp(points, gridf, pc_mean, W1p, b1[None, :], W2, b2[None, :])

    sums = jax.ops.segment_sum(pt_fea, coors_inv, num_segments=_N_VOXELS)
    features = sums / jnp.clip(acc[:, 3:4], 1.0)

    partial_feature = features[unmask_index]
    partial_coors = coors[unmask_index]
    voxel_features_all_one = jnp.ones((coors.shape[0], 1), dtype=jnp.float32)
    return (features, partial_feature, partial_coors, voxel_features_all_one)


# speedup vs baseline: 1.1704x; 1.1704x over previous
import jax
import jax.numpy as jnp
import numpy as np
from jax.experimental import pallas as pl
from jax.experimental.pallas import tpu as pltpu

_N_POINTS = 1000000
_N_VOXELS = 120000
_OUT_CH = 64
_LO = np.array([-51.2, -51.2, -4.0], dtype=np.float32)
_INTERVALS = np.array([102.4 / 512.0, 102.4 / 512.0, 6.4 / 32.0], dtype=np.float32)

_TILE = 2000


def _mlp_body(pts_ref, gc_ref, mean_ref, w1_ref, b1_ref, w2_ref, b2_ref, out_ref):
    pts = pts_ref[...]
    xyz = pts[:, :3]
    t = pts.shape[0]
    neg_lo = jnp.concatenate(
        [jnp.full((t, 1), 51.2, jnp.float32), jnp.full((t, 1), 51.2, jnp.float32),
         jnp.full((t, 1), 4.0, jnp.float32)], axis=1)
    nor = xyz - mean_ref[...]
    ctp = xyz - gc_ref[...] * 0.2 + neg_lo
    zeros = jnp.zeros((pts.shape[0], 6), dtype=jnp.float32)
    pcf = jnp.concatenate([pts, nor, ctp, zeros], axis=1)
    h = jnp.maximum(
        jnp.dot(pcf, w1_ref[...], preferred_element_type=jnp.float32) + b1_ref[...],
        0.0,
    )
    out_ref[...] = (
        jnp.dot(h, w2_ref[...], preferred_element_type=jnp.float32) + b2_ref[...]
    )


def _run_mlp(points, gridf, pc_mean, W1p, b1, W2, b2):
    grid = _N_POINTS // _TILE
    return pl.pallas_call(
        _mlp_body,
        grid=(grid,),
        in_specs=[
            pl.BlockSpec((_TILE, 4), lambda i: (i, 0)),
            pl.BlockSpec((_TILE, 3), lambda i: (i, 0)),
            pl.BlockSpec((_TILE, 3), lambda i: (i, 0)),
            pl.BlockSpec((16, _OUT_CH), lambda i: (0, 0)),
            pl.BlockSpec((1, _OUT_CH), lambda i: (0, 0)),
            pl.BlockSpec((_OUT_CH, _OUT_CH), lambda i: (0, 0)),
            pl.BlockSpec((1, _OUT_CH), lambda i: (0, 0)),
        ],
        out_specs=pl.BlockSpec((_TILE, _OUT_CH), lambda i: (i, 0)),
        out_shape=jax.ShapeDtypeStruct((_N_POINTS, _OUT_CH), jnp.float32),
    )(points, gridf, pc_mean, W1p, b1, W2, b2)


def kernel(points, full_coors, coors_inv, coors, unmask_index, batch_size, W1, b1, W2, b2):
    xyz = points[:, :3]
    ones = jnp.ones((_N_POINTS, 1), dtype=jnp.float32)
    acc = jax.ops.segment_sum(
        jnp.concatenate([xyz, ones], axis=1), coors_inv, num_segments=_N_VOXELS
    )
    vox_mean = acc[:, :3] / jnp.clip(acc[:, 3:4], 1.0)
    pc_mean = vox_mean[coors_inv]

    gridf = full_coors[:, 1:].astype(jnp.float32)
    W1p = jnp.pad(W1, ((0, 6), (0, 0)))
    pt_fea = _run_mlp(points, gridf, pc_mean, W1p, b1[None, :], W2, b2[None, :])

    sums = jax.ops.segment_sum(pt_fea, coors_inv, num_segments=_N_VOXELS)
    features = sums / jnp.clip(acc[:, 3:4], 1.0)

    partial_feature = features[unmask_index]
    partial_coors = coors[unmask_index]
    voxel_features_all_one = jnp.ones((coors.shape[0], 1), dtype=jnp.float32)
    return (features, partial_feature, partial_coors, voxel_features_all_one)


# speedup vs baseline: 1.3005x; 1.1111x over previous
import functools

import jax
import jax.numpy as jnp
from jax import lax
from jax.experimental import pallas as pl
from jax.experimental.pallas import tpu as pltpu
from jax.experimental.pallas import tpu_sc as plsc

_N_POINTS = 1_000_000
_NP_PAD = 1_048_576
_N_VOXELS = 120_000
_NV_PAD = 120_832
_OUT_CH = 64
_NU = 60_000
_NU_PAD = 65_536

_NC, _NS = 2, 16
_BLK = 2048
_IDR = _BLK // 128
_ROWS_PER_TILE = _NV_PAD // _NS

_mesh = functools.partial(
    plsc.VectorSubcoreMesh,
    core_axis_name="c",
    subcore_axis_name="s",
    num_cores=_NC,
    num_subcores=_NS,
)
_sc_params = functools.partial(
    pltpu.CompilerParams, use_tc_tiling_on_sc=False)


def _p12_body(pts1_hbm, ids_hbm, zeros4_hbm, meta_hbm, acc_hbm,
              pbuf, idbuf, gbuf, acc_sh, sem):
    c = lax.axis_index("c")
    s = lax.axis_index("s")
    wid = s * _NC + c
    r0 = s * _ROWS_PER_TILE
    pltpu.sync_copy(zeros4_hbm.at[pl.ds(r0, _ROWS_PER_TILE)],
                    acc_sh.at[pl.ds(r0, _ROWS_PER_TILE)])
    plsc.subcore_barrier()
    @pl.loop(0, _NP_PAD // _NS // 128)
    def _scatter(j):
        pbase = s * (_NP_PAD // _NS) + j * 128
        pltpu.sync_copy(pts1_hbm.at[pl.ds(pbase, 128)], pbuf)
        pltpu.sync_copy(ids_hbm.at[pl.ds(pbase, 128)], idbuf)
        pltpu.sync_copy(pbuf, acc_sh.at[idbuf], add=True)
    plsc.subcore_barrier()
    @pl.when(c == 0)
    def _():
        pltpu.sync_copy(acc_sh.at[pl.ds(r0, _ROWS_PER_TILE)],
                        acc_hbm.at[pl.ds(r0, _ROWS_PER_TILE)])
    @pl.loop(0, _NP_PAD // (_NC * _NS) // 128)
    def _gather(j):
        pbase = wid * (_NP_PAD // (_NC * _NS)) + j * 128
        pltpu.sync_copy(ids_hbm.at[pl.ds(pbase, 128)], idbuf)
        pltpu.async_copy(acc_sh.at[idbuf], gbuf, sem).wait()
        pltpu.sync_copy(gbuf, meta_hbm.at[pl.ds(pbase, 128)])


def _run_p12(pts1, ids_pad, zeros4):
    return pl.kernel(
        _p12_body,
        out_type=(
            jax.ShapeDtypeStruct((_NP_PAD, 8), jnp.float32),
            jax.ShapeDtypeStruct((_NV_PAD, 8), jnp.float32),
        ),
        mesh=_mesh(),
        compiler_params=_sc_params(),
        scratch_types=[
            pltpu.VMEM((128, 8), jnp.float32),
            pltpu.VMEM((128,), jnp.int32),
            pltpu.VMEM((128, 8), jnp.float32),
            pltpu.VMEM_SHARED((_NV_PAD, 8), jnp.float32),
            pltpu.SemaphoreType.DMA,
        ],
    )(pts1, ids_pad, zeros4)


def _p4_body(h_hbm, ids_hbm, zeros8_hbm, hsum_hbm, hbuf, idbuf, acc_sh, sem):
    c = lax.axis_index("c")
    s = lax.axis_index("s")
    r0 = s * _ROWS_PER_TILE
    for g in range(4):
        col = (c * 4 + g) * 8
        pltpu.sync_copy(zeros8_hbm.at[pl.ds(r0, _ROWS_PER_TILE)],
                        acc_sh.at[pl.ds(r0, _ROWS_PER_TILE)])
        plsc.subcore_barrier()

        @pl.loop(0, _NP_PAD // _NS // 128)
        def _scat(j):
            pbase = s * (_NP_PAD // _NS) + j * 128
            pltpu.sync_copy(h_hbm.at[pl.ds(pbase, 128), pl.ds(col, 8)], hbuf)
            pltpu.sync_copy(ids_hbm.at[pl.ds(pbase, 128)], idbuf)
            pltpu.sync_copy(hbuf, acc_sh.at[idbuf], add=True)
        plsc.subcore_barrier()
        pltpu.sync_copy(acc_sh.at[pl.ds(r0, _ROWS_PER_TILE)],
                        hsum_hbm.at[pl.ds(r0, _ROWS_PER_TILE), pl.ds(col, 8)])
        plsc.subcore_barrier()


def _run_p4(h, ids_pad, zeros8):
    return pl.kernel(
        _p4_body,
        out_type=jax.ShapeDtypeStruct((_NV_PAD, _OUT_CH), jnp.float32),
        mesh=_mesh(),
        compiler_params=_sc_params(),
        scratch_types=[
            pltpu.VMEM((128, 8), jnp.float32),
            pltpu.VMEM((128,), jnp.int32),
            pltpu.VMEM_SHARED((_NV_PAD, 8), jnp.float32),
            pltpu.SemaphoreType.DMA,
        ],
    )(h, ids_pad, zeros8)


_P6_BLK = 1024


def _p6_body(feat_hbm, coors_hbm, uidx_hbm, pf_hbm, pc_hbm,
             ubuf, fbuf, cbuf, sem):
    c = lax.axis_index("c")
    s = lax.axis_index("s")
    wid = s * _NC + c
    per_w = _NU_PAD // (_NC * _NS)

    @pl.loop(0, per_w // 128)
    def _gat(j):
        base = wid * per_w + j * 128
        pltpu.sync_copy(uidx_hbm.at[pl.ds(base, 128)], ubuf)
        pltpu.async_copy(feat_hbm.at[ubuf], fbuf, sem).wait()
        pltpu.sync_copy(fbuf, pf_hbm.at[pl.ds(base, 128)])
        pltpu.async_copy(coors_hbm.at[ubuf], cbuf, sem).wait()
        pltpu.sync_copy(cbuf, pc_hbm.at[pl.ds(base, 128)])


def _run_p6(features, coors8, uidx_pad):
    return pl.kernel(
        _p6_body,
        out_type=(
            jax.ShapeDtypeStruct((_NU_PAD, _OUT_CH), jnp.float32),
            jax.ShapeDtypeStruct((_NU_PAD, 8), jnp.int32),
        ),
        mesh=_mesh(),
        compiler_params=_sc_params(),
        scratch_types=[
            pltpu.VMEM((128,), jnp.int32),
            pltpu.VMEM((128, _OUT_CH), jnp.float32),
            pltpu.VMEM((128, 8), jnp.int32),
            pltpu.SemaphoreType.DMA,
        ],
    )(features, coors8, uidx_pad)


def _p3_body(pts_ref, gc_ref, meta_ref, w1_ref, b1_ref, out_ref):
    i = pl.program_id(0)
    pts = pts_ref[...]
    xyz = pts[:, :3]
    t = pts.shape[0]
    meta = meta_ref[...]
    pc_mean = meta[:, :3] / jnp.maximum(meta[:, 3:4], 1.0)
    neg_lo = jnp.concatenate(
        [jnp.full((t, 1), 51.2, jnp.float32), jnp.full((t, 1), 51.2, jnp.float32),
         jnp.full((t, 1), 4.0, jnp.float32)], axis=1)
    nor = xyz - pc_mean
    ctp = xyz - gc_ref[...] * 0.2 + neg_lo
    zeros = jnp.zeros((t, 6), dtype=jnp.float32)
    pcf = jnp.concatenate([pts, nor, ctp, zeros], axis=1)
    h = jnp.maximum(
        jnp.dot(pcf, w1_ref[...], preferred_element_type=jnp.float32) + b1_ref[...],
        0.0,
    )
    row = i * t + lax.broadcasted_iota(jnp.int32, (t, 1), 0)
    out_ref[...] = jnp.where(row < _N_POINTS, h, 0.0)


def _run_p3(points_pad, gridf_pad, meta, W1p, b1):
    grid = _NP_PAD // _BLK
    return pl.pallas_call(
        _p3_body,
        grid=(grid,),
        in_specs=[
            pl.BlockSpec((_BLK, 4), lambda i: (i, 0)),
            pl.BlockSpec((_BLK, 3), lambda i: (i, 0)),
            pl.BlockSpec((_BLK, 8), lambda i: (i, 0)),
            pl.BlockSpec((16, _OUT_CH), lambda i: (0, 0)),
            pl.BlockSpec((1, _OUT_CH), lambda i: (0, 0)),
        ],
        out_specs=pl.BlockSpec((_BLK, _OUT_CH), lambda i: (i, 0)),
        out_shape=jax.ShapeDtypeStruct((_NP_PAD, _OUT_CH), jnp.float32),
    )(points_pad, gridf_pad, meta, W1p, b1)


def _p5_body(hsum_ref, acc_ref, w2_ref, b2_ref, out_ref):
    cnt = acc_ref[...][:, 3:4]
    mean = hsum_ref[...] / jnp.maximum(cnt, 1.0)
    feat = jnp.dot(mean, w2_ref[...], preferred_element_type=jnp.float32) + b2_ref[...]
    out_ref[...] = jnp.where(cnt > 0.0, feat, 0.0)


def _run_p5(hsum, acc, W2, b2):
    grid = _NV_PAD // _BLK
    return pl.pallas_call(
        _p5_body,
        grid=(grid,),
        in_specs=[
            pl.BlockSpec((_BLK, _OUT_CH), lambda i: (i, 0)),
            pl.BlockSpec((_BLK, 8), lambda i: (i, 0)),
            pl.BlockSpec((_OUT_CH, _OUT_CH), lambda i: (0, 0)),
            pl.BlockSpec((1, _OUT_CH), lambda i: (0, 0)),
        ],
        out_specs=pl.BlockSpec((_BLK, _OUT_CH), lambda i: (i, 0)),
        out_shape=jax.ShapeDtypeStruct((_NV_PAD, _OUT_CH), jnp.float32),
    )(hsum, acc, W2, b2)


def kernel(points, full_coors, coors_inv, coors, unmask_index, batch_size,
           W1, b1, W2, b2):
    f32 = jnp.float32
    npad = _NP_PAD - _N_POINTS
    ids_pad = jnp.concatenate(
        [coors_inv, _N_VOXELS + jnp.arange(npad, dtype=jnp.int32) % (_NV_PAD - _N_VOXELS)])
    ids2d = ids_pad.reshape(_NP_PAD // 128, 128)
    pts1 = jnp.concatenate(
        [points[:, :3], jnp.ones((_N_POINTS, 1), f32),
         jnp.zeros((_N_POINTS, 4), f32)], axis=1)
    pts1_pad = jnp.pad(pts1, ((0, npad), (0, 0)))
    zeros4 = jnp.zeros((_NV_PAD, 8), f32)
    zeros8 = jnp.zeros((_NV_PAD, 8), f32)

    meta, acc = _run_p12(pts1_pad, ids_pad, zeros4)

    points_pad = jnp.pad(points, ((0, npad), (0, 0)))
    gridf_pad = jnp.pad(full_coors[:, 1:].astype(f32), ((0, npad), (0, 0)))
    W1p = jnp.pad(W1, ((0, 6), (0, 0)))
    h = _run_p3(points_pad, gridf_pad, meta, W1p, b1[None, :])

    hsum = _run_p4(h, ids_pad, zeros8)
    features_pad = _run_p5(hsum, acc, W2, b2[None, :])
    features = features_pad[:_N_VOXELS]

    upad = _NU_PAD - _NU
    uidx_pad = jnp.concatenate(
        [unmask_index, (jnp.arange(upad, dtype=jnp.int32) * 83) % _N_VOXELS])
    coors8 = jnp.pad(coors, ((0, 0), (0, 4)))
    pf, pc8 = _run_p6(features_pad, coors8, uidx_pad)
    pc = pc8[:, :4]

    partial_feature = pf[:_NU]
    partial_coors = pc[:_NU]
    voxel_features_all_one = jnp.ones((coors.shape[0], 1), f32)
    return (features, partial_feature, partial_coors, voxel_features_all_one)


# speedup vs baseline: 1.7502x; 1.3458x over previous
import functools

import jax
import jax.numpy as jnp
from jax import lax
from jax.experimental import pallas as pl
from jax.experimental.pallas import tpu as pltpu
from jax.experimental.pallas import tpu_sc as plsc

_N_POINTS = 1_000_000
_NP_PAD = 1_048_576
_N_VOXELS = 120_000
_NV_PAD = 120_832
_OUT_CH = 64
_NU = 60_000
_NU_PAD = 65_536

_NC, _NS = 2, 16
_BLK = 2048
_IDR = _BLK // 128
_ROWS_PER_TILE = _NV_PAD // _NS

_mesh = functools.partial(
    plsc.VectorSubcoreMesh,
    core_axis_name="c",
    subcore_axis_name="s",
    num_cores=_NC,
    num_subcores=_NS,
)
_sc_params = functools.partial(
    pltpu.CompilerParams, use_tc_tiling_on_sc=False)


def _p12_body(pts1_hbm, ids_hbm, zeros4_hbm, meta_hbm, acc_hbm,
              pbuf, idbuf, gbuf, acc_sh, sem):
    c = lax.axis_index("c")
    s = lax.axis_index("s")
    wid = s * _NC + c
    r0 = s * _ROWS_PER_TILE
    pltpu.sync_copy(zeros4_hbm.at[pl.ds(r0, _ROWS_PER_TILE)],
                    acc_sh.at[pl.ds(r0, _ROWS_PER_TILE)])
    plsc.subcore_barrier()
    @pl.loop(0, _NP_PAD // _NS // _BLK)
    def _scatter(j):
        pbase = s * (_NP_PAD // _NS) + j * _BLK
        pltpu.sync_copy(pts1_hbm.at[pl.ds(pbase, _BLK)], pbuf)
        pltpu.sync_copy(ids_hbm.at[pl.ds(pbase, _BLK)], idbuf)
        pltpu.sync_copy(pbuf, acc_sh.at[idbuf], add=True)
    plsc.subcore_barrier()
    @pl.when(c == 0)
    def _():
        pltpu.sync_copy(acc_sh.at[pl.ds(r0, _ROWS_PER_TILE)],
                        acc_hbm.at[pl.ds(r0, _ROWS_PER_TILE)])
    @pl.loop(0, _NP_PAD // (_NC * _NS) // _BLK)
    def _gather(j):
        pbase = wid * (_NP_PAD // (_NC * _NS)) + j * _BLK
        pltpu.sync_copy(ids_hbm.at[pl.ds(pbase, _BLK)], idbuf)
        pltpu.async_copy(acc_sh.at[idbuf], gbuf, sem).wait()
        pltpu.sync_copy(gbuf, meta_hbm.at[pl.ds(pbase, _BLK)])


def _run_p12(pts1, ids_pad, zeros4):
    return pl.kernel(
        _p12_body,
        out_type=(
            jax.ShapeDtypeStruct((_NP_PAD, 8), jnp.float32),
            jax.ShapeDtypeStruct((_NV_PAD, 8), jnp.float32),
        ),
        mesh=_mesh(),
        compiler_params=_sc_params(),
        scratch_types=[
            pltpu.VMEM((_BLK, 8), jnp.float32),
            pltpu.VMEM((_BLK,), jnp.int32),
            pltpu.VMEM((_BLK, 8), jnp.float32),
            pltpu.VMEM_SHARED((_NV_PAD, 8), jnp.float32),
            pltpu.SemaphoreType.DMA,
        ],
    )(pts1, ids_pad, zeros4)


def _p4_body(h_hbm, ids_hbm, zeros8_hbm, hsum_hbm, hbuf, idbuf, acc_sh, sem):
    c = lax.axis_index("c")
    s = lax.axis_index("s")
    r0 = s * _ROWS_PER_TILE
    for g in range(4):
        col = (c * 4 + g) * 8
        pltpu.sync_copy(zeros8_hbm.at[pl.ds(r0, _ROWS_PER_TILE)],
                        acc_sh.at[pl.ds(r0, _ROWS_PER_TILE)])
        plsc.subcore_barrier()

        @pl.loop(0, _NP_PAD // _NS // _BLK)
        def _scat(j):
            pbase = s * (_NP_PAD // _NS) + j * _BLK
            pltpu.sync_copy(h_hbm.at[pl.ds(pbase, _BLK), pl.ds(col, 8)], hbuf)
            pltpu.sync_copy(ids_hbm.at[pl.ds(pbase, _BLK)], idbuf)
            pltpu.sync_copy(hbuf, acc_sh.at[idbuf], add=True)
        plsc.subcore_barrier()
        pltpu.sync_copy(acc_sh.at[pl.ds(r0, _ROWS_PER_TILE)],
                        hsum_hbm.at[pl.ds(r0, _ROWS_PER_TILE), pl.ds(col, 8)])
        plsc.subcore_barrier()


def _run_p4(h, ids_pad, zeros8):
    return pl.kernel(
        _p4_body,
        out_type=jax.ShapeDtypeStruct((_NV_PAD, _OUT_CH), jnp.float32),
        mesh=_mesh(),
        compiler_params=_sc_params(),
        scratch_types=[
            pltpu.VMEM((_BLK, 8), jnp.float32),
            pltpu.VMEM((_BLK,), jnp.int32),
            pltpu.VMEM_SHARED((_NV_PAD, 8), jnp.float32),
            pltpu.SemaphoreType.DMA,
        ],
    )(h, ids_pad, zeros8)


_P6_BLK = 1024


def _p6_body(feat_hbm, coors_hbm, uidx_hbm, pf_hbm, pc_hbm,
             ubuf, fbuf, cbuf, sem):
    c = lax.axis_index("c")
    s = lax.axis_index("s")
    wid = s * _NC + c
    per_w = _NU_PAD // (_NC * _NS)

    @pl.loop(0, per_w // _P6_BLK)
    def _gat(j):
        base = wid * per_w + j * _P6_BLK
        pltpu.sync_copy(uidx_hbm.at[pl.ds(base, _P6_BLK)], ubuf)
        pltpu.async_copy(feat_hbm.at[ubuf], fbuf, sem).wait()
        pltpu.sync_copy(fbuf, pf_hbm.at[pl.ds(base, _P6_BLK)])
        pltpu.async_copy(coors_hbm.at[ubuf], cbuf, sem).wait()
        pltpu.sync_copy(cbuf, pc_hbm.at[pl.ds(base, _P6_BLK)])


def _run_p6(features, coors8, uidx_pad):
    return pl.kernel(
        _p6_body,
        out_type=(
            jax.ShapeDtypeStruct((_NU_PAD, _OUT_CH), jnp.float32),
            jax.ShapeDtypeStruct((_NU_PAD, 8), jnp.int32),
        ),
        mesh=_mesh(),
        compiler_params=_sc_params(),
        scratch_types=[
            pltpu.VMEM((_P6_BLK,), jnp.int32),
            pltpu.VMEM((_P6_BLK, _OUT_CH), jnp.float32),
            pltpu.VMEM((_P6_BLK, 8), jnp.int32),
            pltpu.SemaphoreType.DMA,
        ],
    )(features, coors8, uidx_pad)


def _p3_body(pts_ref, gc_ref, meta_ref, w1_ref, b1_ref, out_ref):
    i = pl.program_id(0)
    pts = pts_ref[...]
    xyz = pts[:, :3]
    t = pts.shape[0]
    meta = meta_ref[...]
    pc_mean = meta[:, :3] / jnp.maximum(meta[:, 3:4], 1.0)
    neg_lo = jnp.concatenate(
        [jnp.full((t, 1), 51.2, jnp.float32), jnp.full((t, 1), 51.2, jnp.float32),
         jnp.full((t, 1), 4.0, jnp.float32)], axis=1)
    nor = xyz - pc_mean
    ctp = xyz - gc_ref[...] * 0.2 + neg_lo
    zeros = jnp.zeros((t, 6), dtype=jnp.float32)
    pcf = jnp.concatenate([pts, nor, ctp, zeros], axis=1)
    h = jnp.maximum(
        jnp.dot(pcf, w1_ref[...], preferred_element_type=jnp.float32) + b1_ref[...],
        0.0,
    )
    row = i * t + lax.broadcasted_iota(jnp.int32, (t, 1), 0)
    out_ref[...] = jnp.where(row < _N_POINTS, h, 0.0)


def _run_p3(points_pad, gridf_pad, meta, W1p, b1):
    grid = _NP_PAD // _BLK
    return pl.pallas_call(
        _p3_body,
        grid=(grid,),
        in_specs=[
            pl.BlockSpec((_BLK, 4), lambda i: (i, 0)),
            pl.BlockSpec((_BLK, 3), lambda i: (i, 0)),
            pl.BlockSpec((_BLK, 8), lambda i: (i, 0)),
            pl.BlockSpec((16, _OUT_CH), lambda i: (0, 0)),
            pl.BlockSpec((1, _OUT_CH), lambda i: (0, 0)),
        ],
        out_specs=pl.BlockSpec((_BLK, _OUT_CH), lambda i: (i, 0)),
        out_shape=jax.ShapeDtypeStruct((_NP_PAD, _OUT_CH), jnp.float32),
    )(points_pad, gridf_pad, meta, W1p, b1)


def _p5_body(hsum_ref, acc_ref, w2_ref, b2_ref, out_ref):
    cnt = acc_ref[...][:, 3:4]
    mean = hsum_ref[...] / jnp.maximum(cnt, 1.0)
    feat = jnp.dot(mean, w2_ref[...], preferred_element_type=jnp.float32) + b2_ref[...]
    out_ref[...] = jnp.where(cnt > 0.0, feat, 0.0)


def _run_p5(hsum, acc, W2, b2):
    grid = _NV_PAD // _BLK
    return pl.pallas_call(
        _p5_body,
        grid=(grid,),
        in_specs=[
            pl.BlockSpec((_BLK, _OUT_CH), lambda i: (i, 0)),
            pl.BlockSpec((_BLK, 8), lambda i: (i, 0)),
            pl.BlockSpec((_OUT_CH, _OUT_CH), lambda i: (0, 0)),
            pl.BlockSpec((1, _OUT_CH), lambda i: (0, 0)),
        ],
        out_specs=pl.BlockSpec((_BLK, _OUT_CH), lambda i: (i, 0)),
        out_shape=jax.ShapeDtypeStruct((_NV_PAD, _OUT_CH), jnp.float32),
    )(hsum, acc, W2, b2)


def kernel(points, full_coors, coors_inv, coors, unmask_index, batch_size,
           W1, b1, W2, b2):
    f32 = jnp.float32
    npad = _NP_PAD - _N_POINTS
    ids_pad = jnp.concatenate(
        [coors_inv, _N_VOXELS + jnp.arange(npad, dtype=jnp.int32) % (_NV_PAD - _N_VOXELS)])
    ids2d = ids_pad.reshape(_NP_PAD // 128, 128)
    pts1 = jnp.concatenate(
        [points[:, :3], jnp.ones((_N_POINTS, 1), f32),
         jnp.zeros((_N_POINTS, 4), f32)], axis=1)
    pts1_pad = jnp.pad(pts1, ((0, npad), (0, 0)))
    zeros4 = jnp.zeros((_NV_PAD, 8), f32)
    zeros8 = jnp.zeros((_NV_PAD, 8), f32)

    meta, acc = _run_p12(pts1_pad, ids_pad, zeros4)

    points_pad = jnp.pad(points, ((0, npad), (0, 0)))
    gridf_pad = jnp.pad(full_coors[:, 1:].astype(f32), ((0, npad), (0, 0)))
    W1p = jnp.pad(W1, ((0, 6), (0, 0)))
    h = _run_p3(points_pad, gridf_pad, meta, W1p, b1[None, :])

    hsum = _run_p4(h, ids_pad, zeros8)
    features_pad = _run_p5(hsum, acc, W2, b2[None, :])
    features = features_pad[:_N_VOXELS]

    upad = _NU_PAD - _NU
    uidx_pad = jnp.concatenate(
        [unmask_index, (jnp.arange(upad, dtype=jnp.int32) * 83) % _N_VOXELS])
    coors8 = jnp.pad(coors, ((0, 0), (0, 4)))
    pf, pc8 = _run_p6(features_pad, coors8, uidx_pad)
    pc = pc8[:, :4]

    partial_feature = pf[:_NU]
    partial_coors = pc[:_NU]
    voxel_features_all_one = jnp.ones((coors.shape[0], 1), f32)
    return (features, partial_feature, partial_coors, voxel_features_all_one)


# speedup vs baseline: 1.9859x; 1.1347x over previous
import functools

import jax
import jax.numpy as jnp
from jax import lax
from jax.experimental import pallas as pl
from jax.experimental.pallas import tpu as pltpu
from jax.experimental.pallas import tpu_sc as plsc

_N_POINTS = 1_000_000
_NP_PAD = 1_048_576
_NP2 = _NP_PAD // 2
_N_VOXELS = 120_000
_NV_PAD = 120_832
_NV2 = _NV_PAD // 2
_OUT_CH = 64
_NU = 60_000
_NU_PAD = 65_536

_NC, _NS = 2, 16
_BLK = 2048
_ROWS_PER_TILE = _NV_PAD // _NS

_mesh = functools.partial(
    plsc.VectorSubcoreMesh,
    core_axis_name="c",
    subcore_axis_name="s",
    num_cores=_NC,
    num_subcores=_NS,
)
_sc_params = functools.partial(
    pltpu.CompilerParams, use_tc_tiling_on_sc=False)


def _p12_body(pts1_hbm, ids_hbm, idsp_hbm, zeros8_hbm, meta_hbm, acc_hbm,
              pbuf, idbuf, gbuf, acc_sh, sem):
    c = lax.axis_index("c")
    s = lax.axis_index("s")
    wid = s * _NC + c
    r0 = s * _ROWS_PER_TILE
    pltpu.sync_copy(zeros8_hbm.at[pl.ds(r0, _ROWS_PER_TILE)],
                    acc_sh.at[pl.ds(r0, _ROWS_PER_TILE)])
    plsc.subcore_barrier()
    @pl.loop(0, _NP_PAD // _NS // _BLK)
    def _scatter(j):
        pbase = s * (_NP_PAD // _NS) + j * _BLK
        pltpu.sync_copy(pts1_hbm.at[pl.ds(pbase, _BLK)], pbuf)
        pltpu.sync_copy(ids_hbm.at[pl.ds(pbase, _BLK)], idbuf)
        pltpu.sync_copy(pbuf, acc_sh.at[idbuf], add=True)
    plsc.subcore_barrier()
    @pl.when(c == 0)
    def _():
        pltpu.sync_copy(acc_sh.at[pl.ds(r0, _ROWS_PER_TILE)],
                        acc_hbm.at[pl.ds(r0, _ROWS_PER_TILE)])
    @pl.loop(0, _NP_PAD // (_NC * _NS) // _BLK)
    def _gather(j):
        pbase = wid * (_NP_PAD // (_NC * _NS)) + j * _BLK
        pltpu.sync_copy(idsp_hbm.at[pl.ds(pbase, _BLK)], idbuf)
        pltpu.async_copy(acc_sh.at[idbuf], gbuf, sem).wait()
        pltpu.sync_copy(gbuf, meta_hbm.at[pl.ds(pbase, _BLK)])


def _run_p12(pts1, ids_pad, ids_perm, zeros8):
    return pl.kernel(
        _p12_body,
        out_type=(
            jax.ShapeDtypeStruct((_NP_PAD, 8), jnp.float32),
            jax.ShapeDtypeStruct((_NV_PAD, 8), jnp.float32),
        ),
        mesh=_mesh(),
        compiler_params=_sc_params(),
        scratch_types=[
            pltpu.VMEM((_BLK, 8), jnp.float32),
            pltpu.VMEM((_BLK,), jnp.int32),
            pltpu.VMEM((_BLK, 8), jnp.float32),
            pltpu.VMEM_SHARED((_NV_PAD, 8), jnp.float32),
            pltpu.SemaphoreType.DMA,
        ],
    )(pts1, ids_pad, ids_perm, zeros8)


def _p4_body(h2_hbm, ids_hbm, zeros8_hbm, hsum2_hbm, hbuf, idbuf, acc_sh, sem):
    c = lax.axis_index("c")
    s = lax.axis_index("s")
    r0 = s * _ROWS_PER_TILE
    half_out = s // 8
    row0 = r0 - half_out * _NV2
    for g in range(4):
        col_g = (c * 4 + g) * 8
        pltpu.sync_copy(zeros8_hbm.at[pl.ds(r0, _ROWS_PER_TILE)],
                        acc_sh.at[pl.ds(r0, _ROWS_PER_TILE)])
        plsc.subcore_barrier()
        for hh in range(2):
            col = hh * 64 + col_g

            @pl.loop(0, _NP2 // _NS // _BLK)
            def _scat(j):
                rbase = s * (_NP2 // _NS) + j * _BLK
                pltpu.sync_copy(h2_hbm.at[pl.ds(rbase, _BLK), pl.ds(col, 8)],
                                hbuf)
                pltpu.sync_copy(ids_hbm.at[pl.ds(hh * _NP2 + rbase, _BLK)],
                                idbuf)
                pltpu.sync_copy(hbuf, acc_sh.at[idbuf], add=True)
        plsc.subcore_barrier()
        pltpu.sync_copy(
            acc_sh.at[pl.ds(r0, _ROWS_PER_TILE)],
            hsum2_hbm.at[pl.ds(row0, _ROWS_PER_TILE),
                         pl.ds(half_out * 64 + col_g, 8)])
        plsc.subcore_barrier()


def _run_p4(h2, ids_pad, zeros8):
    return pl.kernel(
        _p4_body,
        out_type=jax.ShapeDtypeStruct((_NV2, 128), jnp.float32),
        mesh=_mesh(),
        compiler_params=_sc_params(),
        scratch_types=[
            pltpu.VMEM((_BLK, 8), jnp.float32),
            pltpu.VMEM((_BLK,), jnp.int32),
            pltpu.VMEM_SHARED((_NV_PAD, 8), jnp.float32),
            pltpu.SemaphoreType.DMA,
        ],
    )(h2, ids_pad, zeros8)


_P6_BLK = 1024


def _p6_body(feat_hbm, coors_hbm, uidx_hbm, pf_hbm, pc_hbm,
             ubuf, fbuf, cbuf, sem):
    c = lax.axis_index("c")
    s = lax.axis_index("s")
    wid = s * _NC + c
    per_w = _NU_PAD // (_NC * _NS)

    @pl.loop(0, per_w // _P6_BLK)
    def _gat(j):
        base = wid * per_w + j * _P6_BLK
        pltpu.sync_copy(uidx_hbm.at[pl.ds(base, _P6_BLK)], ubuf)
        pltpu.async_copy(feat_hbm.at[ubuf], fbuf, sem).wait()
        pltpu.sync_copy(fbuf, pf_hbm.at[pl.ds(base, _P6_BLK)])
        pltpu.async_copy(coors_hbm.at[ubuf], cbuf, sem).wait()
        pltpu.sync_copy(cbuf, pc_hbm.at[pl.ds(base, _P6_BLK)])


def _run_p6(features, coors8, uidx_pad):
    return pl.kernel(
        _p6_body,
        out_type=(
            jax.ShapeDtypeStruct((_NU_PAD, _OUT_CH), jnp.float32),
            jax.ShapeDtypeStruct((_NU_PAD, 8), jnp.int32),
        ),
        mesh=_mesh(),
        compiler_params=_sc_params(),
        scratch_types=[
            pltpu.VMEM((_P6_BLK,), jnp.int32),
            pltpu.VMEM((_P6_BLK, _OUT_CH), jnp.float32),
            pltpu.VMEM((_P6_BLK, 8), jnp.int32),
            pltpu.SemaphoreType.DMA,
        ],
    )(features, coors8, uidx_pad)


def _half_h(pts, gc, meta8, w1, b1):
    xyz = pts[:, :3]
    t = pts.shape[0]
    pc_mean = meta8[:, :3] / jnp.maximum(meta8[:, 3:4], 1.0)
    neg_lo = jnp.concatenate(
        [jnp.full((t, 1), 51.2, jnp.float32), jnp.full((t, 1), 51.2, jnp.float32),
         jnp.full((t, 1), 4.0, jnp.float32)], axis=1)
    nor = xyz - pc_mean
    ctp = xyz - gc * 0.2 + neg_lo
    zeros = jnp.zeros((t, 6), dtype=jnp.float32)
    pcf = jnp.concatenate([pts, nor, ctp, zeros], axis=1)
    return jnp.maximum(
        jnp.dot(pcf, w1, preferred_element_type=jnp.float32) + b1, 0.0)


def _p3_body(pts_lo, pts_hi, gc_lo, gc_hi, meta_lo, meta_hi,
             w1_ref, b1_ref, out_ref):
    i = pl.program_id(0)
    w1 = w1_ref[...]
    b1 = b1_ref[...]
    mb_lo = meta_lo[...]
    mb_hi = meta_hi[...]
    m_lo = jnp.concatenate([mb_lo[:, 8 * j:8 * j + 8] for j in range(16)], axis=0)
    m_hi = jnp.concatenate([mb_hi[:, 8 * j:8 * j + 8] for j in range(16)], axis=0)
    h_lo = _half_h(pts_lo[...], gc_lo[...], m_lo, w1, b1)
    h_hi = _half_h(pts_hi[...], gc_hi[...], m_hi, w1, b1)
    row_hi = _NP2 + i * _BLK + lax.broadcasted_iota(jnp.int32, (_BLK, 1), 0)
    h_hi = jnp.where(row_hi < _N_POINTS, h_hi, 0.0)
    out_ref[...] = jnp.concatenate([h_lo, h_hi], axis=1)


def _run_p3(points_pad, gridf_pad, meta128, W1p, b1):
    grid = _NP2 // _BLK
    nb = grid
    return pl.pallas_call(
        _p3_body,
        grid=(grid,),
        in_specs=[
            pl.BlockSpec((_BLK, 4), lambda i: (i, 0)),
            pl.BlockSpec((_BLK, 4), lambda i: (i + nb, 0)),
            pl.BlockSpec((_BLK, 3), lambda i: (i, 0)),
            pl.BlockSpec((_BLK, 3), lambda i: (i + nb, 0)),
            pl.BlockSpec((_BLK // 16, 128), lambda i: (i, 0)),
            pl.BlockSpec((_BLK // 16, 128), lambda i: (i + nb, 0)),
            pl.BlockSpec((16, _OUT_CH), lambda i: (0, 0)),
            pl.BlockSpec((1, _OUT_CH), lambda i: (0, 0)),
        ],
        out_specs=pl.BlockSpec((_BLK, 128), lambda i: (i, 0)),
        out_shape=jax.ShapeDtypeStruct((_NP2, 128), jnp.float32),
    )(points_pad, points_pad, gridf_pad, gridf_pad, meta128, meta128, W1p, b1)


_P5_BLK = 1024


def _p5_body(hsum_ref, acc_ref, w2_ref, b2_ref, out_ref):
    i = pl.program_id(0)
    half = i // (_NV2 // _P5_BLK)
    hs = hsum_ref[...]
    hsum = jnp.where(half == 0, hs[:, :64], hs[:, 64:])
    cnt = acc_ref[...][:, 3:4]
    mean = hsum / jnp.maximum(cnt, 1.0)
    feat = jnp.dot(mean, w2_ref[...], preferred_element_type=jnp.float32) + b2_ref[...]
    out_ref[...] = jnp.where(cnt > 0.0, feat, 0.0)


def _run_p5(hsum2, acc, W2, b2):
    nb = _NV2 // _P5_BLK
    grid = 2 * nb
    return pl.pallas_call(
        _p5_body,
        grid=(grid,),
        in_specs=[
            pl.BlockSpec((_P5_BLK, 128), lambda i: (i % 59, 0)),
            pl.BlockSpec((_P5_BLK, 8), lambda i: (i, 0)),
            pl.BlockSpec((_OUT_CH, _OUT_CH), lambda i: (0, 0)),
            pl.BlockSpec((1, _OUT_CH), lambda i: (0, 0)),
        ],
        out_specs=pl.BlockSpec((_P5_BLK, _OUT_CH), lambda i: (i, 0)),
        out_shape=jax.ShapeDtypeStruct((_NV_PAD, _OUT_CH), jnp.float32),
    )(hsum2, acc, W2, b2)


def kernel(points, full_coors, coors_inv, coors, unmask_index, batch_size,
           W1, b1, W2, b2):
    f32 = jnp.float32
    npad = _NP_PAD - _N_POINTS
    ids_pad = jnp.concatenate(
        [coors_inv, _N_VOXELS + jnp.arange(npad, dtype=jnp.int32) % (_NV_PAD - _N_VOXELS)])
    pts1 = jnp.concatenate(
        [points[:, :3], jnp.ones((_N_POINTS, 1), f32),
         jnp.zeros((_N_POINTS, 4), f32)], axis=1)
    pts1_pad = jnp.pad(pts1, ((0, npad), (0, 0)))
    zeros8 = jnp.zeros((_NV_PAD, 8), f32)

    ids_perm = ids_pad.reshape(_NP_PAD // _BLK, 16, 128).transpose(0, 2, 1).reshape(_NP_PAD)
    meta, acc = _run_p12(pts1_pad, ids_pad, ids_perm, zeros8)
    meta128 = meta.reshape(_NP_PAD // 16, 128)

    points_pad = jnp.pad(points, ((0, npad), (0, 0)))
    gridf_pad = jnp.pad(full_coors[:, 1:].astype(f32), ((0, npad), (0, 0)))
    W1p = jnp.pad(W1, ((0, 6), (0, 0)))
    h2 = _run_p3(points_pad, gridf_pad, meta128, W1p, b1[None, :])

    hsum2 = _run_p4(h2, ids_pad, zeros8)
    features_pad = _run_p5(hsum2, acc, W2, b2[None, :])
    features = features_pad[:_N_VOXELS]

    upad = _NU_PAD - _NU
    uidx_pad = jnp.concatenate(
        [unmask_index, (jnp.arange(upad, dtype=jnp.int32) * 83) % _N_VOXELS])
    coors8 = jnp.pad(coors, ((0, 0), (0, 4)))
    pf, pc8 = _run_p6(features_pad, coors8, uidx_pad)

    partial_feature = pf[:_NU]
    partial_coors = pc8[:_NU, :4]
    voxel_features_all_one = jnp.ones((coors.shape[0], 1), f32)
    return (features, partial_feature, partial_coors, voxel_features_all_one)


# speedup vs baseline: 4.9019x; 2.4683x over previous
import functools

import jax
import jax.numpy as jnp
from jax import lax
from jax.experimental import pallas as pl
from jax.experimental.pallas import tpu as pltpu
from jax.experimental.pallas import tpu_sc as plsc

_N_POINTS = 1_000_000
_NP_PAD = 1_048_576
_NP2 = _NP_PAD // 2
_N_VOXELS = 120_000
_NV_PAD = 120_832
_NV2 = _NV_PAD // 2
_OUT_CH = 64
_NU = 60_000
_NU_PAD = 65_536

_NC, _NS = 2, 16
_BLK = 2048
_ROWS_PER_TILE = _NV_PAD // _NS

_mesh = functools.partial(
    plsc.VectorSubcoreMesh,
    core_axis_name="c",
    subcore_axis_name="s",
    num_cores=_NC,
    num_subcores=_NS,
)
_sc_params = functools.partial(
    pltpu.CompilerParams, use_tc_tiling_on_sc=False)


def _p12_body(pts8_hbm, idsp_hbm, zeros8_hbm, meta_hbm, acc_hbm,
              pbuf, idbuf, gbuf, acc_sh, sem):
    c = lax.axis_index("c")
    s = lax.axis_index("s")
    wid = s * _NC + c
    r0 = s * _ROWS_PER_TILE
    pltpu.sync_copy(zeros8_hbm.at[pl.ds(r0, _ROWS_PER_TILE)],
                    acc_sh.at[pl.ds(r0, _ROWS_PER_TILE)])
    plsc.subcore_barrier()
    @pl.loop(0, _NP_PAD // _NS // _BLK)
    def _scatter(j):
        pbase = s * (_NP_PAD // _NS) + j * _BLK
        pltpu.sync_copy(pts8_hbm.at[pl.ds(pbase, _BLK)], pbuf)
        pltpu.sync_copy(idsp_hbm.at[pl.ds(pbase, _BLK)], idbuf)
        pltpu.sync_copy(pbuf, acc_sh.at[idbuf], add=True)
    plsc.subcore_barrier()
    @pl.when(c == 0)
    def _():
        pltpu.sync_copy(acc_sh.at[pl.ds(r0, _ROWS_PER_TILE)],
                        acc_hbm.at[pl.ds(r0, _ROWS_PER_TILE)])
    @pl.loop(0, _NP_PAD // (_NC * _NS) // _BLK)
    def _gather(j):
        pbase = wid * (_NP_PAD // (_NC * _NS)) + j * _BLK
        pltpu.sync_copy(idsp_hbm.at[pl.ds(pbase, _BLK)], idbuf)
        pltpu.async_copy(acc_sh.at[idbuf], gbuf, sem).wait()
        pltpu.sync_copy(gbuf, meta_hbm.at[pl.ds(pbase, _BLK)])


def _run_p12(pts8_sc, ids_perm, zeros8):
    return pl.kernel(
        _p12_body,
        out_type=(
            jax.ShapeDtypeStruct((_NP_PAD, 8), jnp.float32),
            jax.ShapeDtypeStruct((_NV_PAD, 8), jnp.float32),
        ),
        mesh=_mesh(),
        compiler_params=_sc_params(),
        scratch_types=[
            pltpu.VMEM((_BLK, 8), jnp.float32),
            pltpu.VMEM((_BLK,), jnp.int32),
            pltpu.VMEM((_BLK, 8), jnp.float32),
            pltpu.VMEM_SHARED((_NV_PAD, 8), jnp.float32),
            pltpu.SemaphoreType.DMA,
        ],
    )(pts8_sc, ids_perm, zeros8)


def _p4_body(h2_hbm, ids_hbm, zeros8_hbm, hsum2_hbm, hbuf, idbuf, acc_sh, sem):
    c = lax.axis_index("c")
    s = lax.axis_index("s")
    r0 = s * _ROWS_PER_TILE
    half_out = s // 8
    row0 = r0 - half_out * _NV2
    for g in range(4):
        col_g = (c * 4 + g) * 8
        pltpu.sync_copy(zeros8_hbm.at[pl.ds(r0, _ROWS_PER_TILE)],
                        acc_sh.at[pl.ds(r0, _ROWS_PER_TILE)])
        plsc.subcore_barrier()
        for hh in range(2):
            col = hh * 64 + col_g

            @pl.loop(0, _NP2 // _NS // _BLK)
            def _scat(j):
                rbase = s * (_NP2 // _NS) + j * _BLK
                pltpu.sync_copy(h2_hbm.at[pl.ds(rbase, _BLK), pl.ds(col, 8)],
                                hbuf)
                pltpu.sync_copy(ids_hbm.at[pl.ds(hh * _NP2 + rbase, _BLK)],
                                idbuf)
                pltpu.sync_copy(hbuf, acc_sh.at[idbuf], add=True)
        plsc.subcore_barrier()
        pltpu.sync_copy(
            acc_sh.at[pl.ds(r0, _ROWS_PER_TILE)],
            hsum2_hbm.at[pl.ds(row0, _ROWS_PER_TILE),
                         pl.ds(half_out * 64 + col_g, 8)])
        plsc.subcore_barrier()


def _run_p4(h2, ids_pad, zeros8):
    return pl.kernel(
        _p4_body,
        out_type=jax.ShapeDtypeStruct((_NV2, 128), jnp.float32),
        mesh=_mesh(),
        compiler_params=_sc_params(),
        scratch_types=[
            pltpu.VMEM((_BLK, 8), jnp.float32),
            pltpu.VMEM((_BLK,), jnp.int32),
            pltpu.VMEM_SHARED((_NV_PAD, 8), jnp.float32),
            pltpu.SemaphoreType.DMA,
        ],
    )(h2, ids_pad, zeros8)


_P6_BLK = 1024


def _p6_body(feat_hbm, coors_hbm, uidx_hbm, pf_hbm, pc_hbm,
             ubuf, fbuf, cbuf, sem):
    c = lax.axis_index("c")
    s = lax.axis_index("s")
    wid = s * _NC + c
    per_w = _NU_PAD // (_NC * _NS)

    @pl.loop(0, per_w // _P6_BLK)
    def _gat(j):
        base = wid * per_w + j * _P6_BLK
        pltpu.sync_copy(uidx_hbm.at[pl.ds(base, _P6_BLK)], ubuf)
        pltpu.async_copy(feat_hbm.at[ubuf], fbuf, sem).wait()
        pltpu.sync_copy(fbuf, pf_hbm.at[pl.ds(base, _P6_BLK)])
        pltpu.async_copy(coors_hbm.at[ubuf], cbuf, sem).wait()
        pltpu.sync_copy(cbuf, pc_hbm.at[pl.ds(base, _P6_BLK)])


def _run_p6(features, coors8, uidx_pad):
    return pl.kernel(
        _p6_body,
        out_type=(
            jax.ShapeDtypeStruct((_NU_PAD, _OUT_CH), jnp.float32),
            jax.ShapeDtypeStruct((_NU_PAD, 8), jnp.int32),
        ),
        mesh=_mesh(),
        compiler_params=_sc_params(),
        scratch_types=[
            pltpu.VMEM((_P6_BLK,), jnp.int32),
            pltpu.VMEM((_P6_BLK, _OUT_CH), jnp.float32),
            pltpu.VMEM((_P6_BLK, 8), jnp.int32),
            pltpu.SemaphoreType.DMA,
        ],
    )(features, coors8, uidx_pad)


def _unpack16(blk):
    return jnp.concatenate([blk[:, 8 * j:8 * j + 8] for j in range(16)], axis=0)


def _half_h(up8, meta8, w1, b1):
    pts = up8[:, :4]
    gc = up8[:, 5:8]
    xyz = up8[:, :3]
    t = up8.shape[0]
    pc_mean = meta8[:, :3] / jnp.maximum(meta8[:, 4:5], 1.0)
    neg_lo = jnp.concatenate(
        [jnp.full((t, 1), 51.2, jnp.float32), jnp.full((t, 1), 51.2, jnp.float32),
         jnp.full((t, 1), 4.0, jnp.float32)], axis=1)
    nor = xyz - pc_mean
    ctp = xyz - gc * 0.2 + neg_lo
    zeros = jnp.zeros((t, 6), dtype=jnp.float32)
    pcf = jnp.concatenate([pts, nor, ctp, zeros], axis=1)
    return jnp.maximum(
        jnp.dot(pcf, w1, preferred_element_type=jnp.float32) + b1, 0.0)


def _p3_body(pts_lo, pts_hi, meta_lo, meta_hi, w1_ref, b1_ref, out_ref):
    w1 = w1_ref[...]
    b1 = b1_ref[...]
    h_lo = _half_h(_unpack16(pts_lo[...]), _unpack16(meta_lo[...]), w1, b1)
    h_hi = _half_h(_unpack16(pts_hi[...]), _unpack16(meta_hi[...]), w1, b1)
    out_ref[...] = jnp.concatenate([h_lo, h_hi], axis=1)


def _run_p3(pts8p, meta128, W1p, b1):
    grid = _NP2 // _BLK
    nb = grid
    return pl.pallas_call(
        _p3_body,
        grid=(grid,),
        in_specs=[
            pl.BlockSpec((_BLK // 16, 128), lambda i: (i, 0)),
            pl.BlockSpec((_BLK // 16, 128), lambda i: (i + nb, 0)),
            pl.BlockSpec((_BLK // 16, 128), lambda i: (i, 0)),
            pl.BlockSpec((_BLK // 16, 128), lambda i: (i + nb, 0)),
            pl.BlockSpec((16, _OUT_CH), lambda i: (0, 0)),
            pl.BlockSpec((1, _OUT_CH), lambda i: (0, 0)),
        ],
        out_specs=pl.BlockSpec((_BLK, 128), lambda i: (i, 0)),
        out_shape=jax.ShapeDtypeStruct((_NP2, 128), jnp.float32),
    )(pts8p, pts8p, meta128, meta128, W1p, b1)


_P5_BLK = 1024


def _p5_body(hsum_ref, acc_ref, w2_ref, b2_ref, out_ref):
    i = pl.program_id(0)
    half = i // (_NV2 // _P5_BLK)
    hs = hsum_ref[...]
    hsum = jnp.where(half == 0, hs[:, :64], hs[:, 64:])
    cnt = acc_ref[...][:, 4:5]
    mean = hsum / jnp.maximum(cnt, 1.0)
    feat = jnp.dot(mean, w2_ref[...], preferred_element_type=jnp.float32) + b2_ref[...]
    out_ref[...] = jnp.where(cnt > 0.0, feat, 0.0)


def _run_p5(hsum2, acc, W2, b2):
    nb = _NV2 // _P5_BLK
    grid = 2 * nb
    return pl.pallas_call(
        _p5_body,
        grid=(grid,),
        in_specs=[
            pl.BlockSpec((_P5_BLK, 128), lambda i: (i % 59, 0)),
            pl.BlockSpec((_P5_BLK, 8), lambda i: (i, 0)),
            pl.BlockSpec((_OUT_CH, _OUT_CH), lambda i: (0, 0)),
            pl.BlockSpec((1, _OUT_CH), lambda i: (0, 0)),
        ],
        out_specs=pl.BlockSpec((_P5_BLK, _OUT_CH), lambda i: (i, 0)),
        out_shape=jax.ShapeDtypeStruct((_NV_PAD, _OUT_CH), jnp.float32),
    )(hsum2, acc, W2, b2)


def kernel(points, full_coors, coors_inv, coors, unmask_index, batch_size,
           W1, b1, W2, b2):
    f32 = jnp.float32
    npad = _NP_PAD - _N_POINTS
    ids_pad = jnp.concatenate(
        [coors_inv, _N_VOXELS + jnp.arange(npad, dtype=jnp.int32) % (_NV_PAD - _N_VOXELS)])
    zeros8 = jnp.zeros((_NV_PAD, 8), f32)
    nb = _NP_PAD // _BLK

    pts8 = jnp.concatenate(
        [points, jnp.ones((_N_POINTS, 1), f32),
         full_coors[:, 1:].astype(f32)], axis=1)
    pts8_pad = jnp.pad(pts8, ((0, npad), (0, 0)))
    pts8p = pts8_pad.reshape(nb, 16, 128, 8).transpose(0, 2, 1, 3).reshape(
        _NP_PAD // 16, 128)
    ids_perm = ids_pad.reshape(nb, 16, 128).transpose(0, 2, 1).reshape(_NP_PAD)

    meta, acc = _run_p12(pts8p.reshape(_NP_PAD, 8), ids_perm, zeros8)
    meta128 = meta.reshape(_NP_PAD // 16, 128)

    W1p = jnp.pad(W1, ((0, 6), (0, 0)))
    h2 = _run_p3(pts8p, meta128, W1p, b1[None, :])

    hsum2 = _run_p4(h2, ids_pad, zeros8)
    features_pad = _run_p5(hsum2, acc, W2, b2[None, :])
    features = features_pad[:_N_VOXELS]

    upad = _NU_PAD - _NU
    uidx_pad = jnp.concatenate(
        [unmask_index, (jnp.arange(upad, dtype=jnp.int32) * 83) % _N_VOXELS])
    coors8 = jnp.pad(coors, ((0, 0), (0, 4)))
    pf, pc8 = _run_p6(features_pad, coors8, uidx_pad)

    partial_feature = pf[:_NU]
    partial_coors = pc8[:_NU, :4]
    voxel_features_all_one = jnp.ones((coors.shape[0], 1), f32)
    return (features, partial_feature, partial_coors, voxel_features_all_one)


# speedup vs baseline: 5.6426x; 1.1511x over previous
import functools

import jax
import jax.numpy as jnp
from jax import lax
from jax.experimental import pallas as pl
from jax.experimental.pallas import tpu as pltpu
from jax.experimental.pallas import tpu_sc as plsc

_N_POINTS = 1_000_000
_NP_PAD = 1_048_576
_NP2 = _NP_PAD // 2
_N_VOXELS = 120_000
_NV_PAD = 120_832
_NV2 = _NV_PAD // 2
_OUT_CH = 64
_NU = 60_000
_NU_PAD = 65_536

_NC, _NS = 2, 16
_BLK = 2048
_ROWS_PER_TILE = _NV_PAD // _NS

_mesh = functools.partial(
    plsc.VectorSubcoreMesh,
    core_axis_name="c",
    subcore_axis_name="s",
    num_cores=_NC,
    num_subcores=_NS,
)
_sc_params = functools.partial(
    pltpu.CompilerParams, use_tc_tiling_on_sc=False)


def _p12_body(pts8_hbm, idsp_hbm, zeros8_hbm, meta_hbm, acc_hbm,
              pbuf, idbuf, gbuf, acc_sh, sem):
    c = lax.axis_index("c")
    s = lax.axis_index("s")
    wid = s * _NC + c
    r0 = s * _ROWS_PER_TILE
    pltpu.sync_copy(zeros8_hbm.at[pl.ds(r0, _ROWS_PER_TILE)],
                    acc_sh.at[pl.ds(r0, _ROWS_PER_TILE)])
    plsc.subcore_barrier()
    @pl.loop(0, _NP_PAD // _NS // _BLK)
    def _scatter(j):
        pbase = s * (_NP_PAD // _NS) + j * _BLK
        pltpu.sync_copy(pts8_hbm.at[pl.ds(pbase, _BLK)], pbuf)
        pltpu.sync_copy(idsp_hbm.at[pl.ds(pbase, _BLK)], idbuf)
        pltpu.sync_copy(pbuf, acc_sh.at[idbuf], add=True)
    plsc.subcore_barrier()
    @pl.when(c == 0)
    def _():
        pltpu.sync_copy(acc_sh.at[pl.ds(r0, _ROWS_PER_TILE)],
                        acc_hbm.at[pl.ds(r0, _ROWS_PER_TILE)])
    @pl.loop(0, _NP_PAD // (_NC * _NS) // _BLK)
    def _gather(j):
        pbase = wid * (_NP_PAD // (_NC * _NS)) + j * _BLK
        pltpu.sync_copy(idsp_hbm.at[pl.ds(pbase, _BLK)], idbuf)
        pltpu.async_copy(acc_sh.at[idbuf], gbuf, sem).wait()
        pltpu.sync_copy(gbuf, meta_hbm.at[pl.ds(pbase, _BLK)])


def _run_p12(pts8_sc, ids_perm, zeros8):
    return pl.kernel(
        _p12_body,
        out_type=(
            jax.ShapeDtypeStruct((_NP_PAD, 8), jnp.float32),
            jax.ShapeDtypeStruct((_NV_PAD, 8), jnp.float32),
        ),
        mesh=_mesh(),
        compiler_params=_sc_params(),
        scratch_types=[
            pltpu.VMEM((_BLK, 8), jnp.float32),
            pltpu.VMEM((_BLK,), jnp.int32),
            pltpu.VMEM((_BLK, 8), jnp.float32),
            pltpu.VMEM_SHARED((_NV_PAD, 8), jnp.float32),
            pltpu.SemaphoreType.DMA,
        ],
    )(pts8_sc, ids_perm, zeros8)


def _p4_body(h2_hbm, ids_hbm, zeros8_hbm, hsum2_hbm, hbuf, idbuf, acc_sh, sem):
    c = lax.axis_index("c")
    s = lax.axis_index("s")
    r0 = s * _ROWS_PER_TILE
    half_out = s // 8
    row0 = r0 - half_out * _NV2
    for g in range(4):
        col_g = (c * 4 + g) * 8
        pltpu.sync_copy(zeros8_hbm.at[pl.ds(r0, _ROWS_PER_TILE)],
                        acc_sh.at[pl.ds(r0, _ROWS_PER_TILE)])
        plsc.subcore_barrier()
        for hh in range(2):
            col = hh * 64 + col_g

            @pl.loop(0, _NP2 // _NS // _BLK)
            def _scat(j):
                rbase = s * (_NP2 // _NS) + j * _BLK
                pltpu.sync_copy(h2_hbm.at[pl.ds(rbase, _BLK), pl.ds(col, 8)],
                                hbuf)
                pltpu.sync_copy(ids_hbm.at[pl.ds(hh * _NP2 + rbase, _BLK)],
                                idbuf)
                pltpu.sync_copy(hbuf, acc_sh.at[idbuf], add=True)
        plsc.subcore_barrier()
        pltpu.sync_copy(
            acc_sh.at[pl.ds(r0, _ROWS_PER_TILE)],
            hsum2_hbm.at[pl.ds(row0, _ROWS_PER_TILE),
                         pl.ds(half_out * 64 + col_g, 8)])
        plsc.subcore_barrier()


def _run_p4(h2, ids_pad, zeros8):
    return pl.kernel(
        _p4_body,
        out_type=jax.ShapeDtypeStruct((_NV2, 128), jnp.float32),
        mesh=_mesh(),
        compiler_params=_sc_params(),
        scratch_types=[
            pltpu.VMEM((_BLK, 8), jnp.float32),
            pltpu.VMEM((_BLK,), jnp.int32),
            pltpu.VMEM_SHARED((_NV_PAD, 8), jnp.float32),
            pltpu.SemaphoreType.DMA,
        ],
    )(h2, ids_pad, zeros8)


_P6_BLK = 1024


def _p6_body(feat_hbm, coors_hbm, uidx_hbm, pf_hbm, pc_hbm,
             ubuf, fbuf, cbuf, sem):
    c = lax.axis_index("c")
    s = lax.axis_index("s")
    wid = s * _NC + c
    per_w = _NU_PAD // (_NC * _NS)

    @pl.loop(0, per_w // _P6_BLK)
    def _gat(j):
        base = wid * per_w + j * _P6_BLK
        pltpu.sync_copy(uidx_hbm.at[pl.ds(base, _P6_BLK)], ubuf)
        pltpu.async_copy(feat_hbm.at[ubuf], fbuf, sem).wait()
        pltpu.sync_copy(fbuf, pf_hbm.at[pl.ds(base, _P6_BLK)])
        pltpu.async_copy(coors_hbm.at[ubuf], cbuf, sem).wait()
        pltpu.sync_copy(cbuf, pc_hbm.at[pl.ds(base, _P6_BLK)])


def _run_p6(features, coors8, uidx_pad):
    return pl.kernel(
        _p6_body,
        out_type=(
            jax.ShapeDtypeStruct((_NU_PAD, _OUT_CH), jnp.float32),
            jax.ShapeDtypeStruct((_NU_PAD, 8), jnp.int32),
        ),
        mesh=_mesh(),
        compiler_params=_sc_params(),
        scratch_types=[
            pltpu.VMEM((_P6_BLK,), jnp.int32),
            pltpu.VMEM((_P6_BLK, _OUT_CH), jnp.float32),
            pltpu.VMEM((_P6_BLK, 8), jnp.int32),
            pltpu.SemaphoreType.DMA,
        ],
    )(features, coors8, uidx_pad)


def _unpack16(blk):
    return jnp.concatenate([blk[:, 8 * j:8 * j + 8] for j in range(16)], axis=0)


def _half_h(up8, meta8, w1a, w1b, b1p):
    mdiv = meta8 / jnp.maximum(meta8[:, 4:5], 1.0)
    z = (jnp.dot(up8, w1a, preferred_element_type=jnp.float32)
         + jnp.dot(mdiv, w1b, preferred_element_type=jnp.float32) + b1p)
    return jnp.maximum(z, 0.0)


def _p3_body(pts_lo, pts_hi, meta_lo, meta_hi, w1a_ref, w1b_ref, b1_ref,
             out_ref):
    w1a = w1a_ref[...]
    w1b = w1b_ref[...]
    b1p = b1_ref[...]
    h_lo = _half_h(_unpack16(pts_lo[...]), _unpack16(meta_lo[...]), w1a, w1b, b1p)
    h_hi = _half_h(_unpack16(pts_hi[...]), _unpack16(meta_hi[...]), w1a, w1b, b1p)
    out_ref[...] = jnp.concatenate([h_lo, h_hi], axis=1)


def _run_p3(pts8p, meta128, W1a, W1b, b1p):
    grid = _NP2 // _BLK
    nb = grid
    return pl.pallas_call(
        _p3_body,
        grid=(grid,),
        in_specs=[
            pl.BlockSpec((_BLK // 16, 128), lambda i: (i, 0)),
            pl.BlockSpec((_BLK // 16, 128), lambda i: (i + nb, 0)),
            pl.BlockSpec((_BLK // 16, 128), lambda i: (i, 0)),
            pl.BlockSpec((_BLK // 16, 128), lambda i: (i + nb, 0)),
            pl.BlockSpec((8, _OUT_CH), lambda i: (0, 0)),
            pl.BlockSpec((8, _OUT_CH), lambda i: (0, 0)),
            pl.BlockSpec((1, _OUT_CH), lambda i: (0, 0)),
        ],
        out_specs=pl.BlockSpec((_BLK, 128), lambda i: (i, 0)),
        out_shape=jax.ShapeDtypeStruct((_NP2, 128), jnp.float32),
    )(pts8p, pts8p, meta128, meta128, W1a, W1b, b1p)


_P5_BLK = 1024


def _p5_body(hsum_ref, acc_ref, w2_ref, b2_ref, out_ref):
    i = pl.program_id(0)
    half = i // (_NV2 // _P5_BLK)
    hs = hsum_ref[...]
    hsum = jnp.where(half == 0, hs[:, :64], hs[:, 64:])
    cnt = acc_ref[...][:, 4:5]
    mean = hsum / jnp.maximum(cnt, 1.0)
    feat = jnp.dot(mean, w2_ref[...], preferred_element_type=jnp.float32) + b2_ref[...]
    out_ref[...] = jnp.where(cnt > 0.0, feat, 0.0)


def _run_p5(hsum2, acc, W2, b2):
    nb = _NV2 // _P5_BLK
    grid = 2 * nb
    return pl.pallas_call(
        _p5_body,
        grid=(grid,),
        in_specs=[
            pl.BlockSpec((_P5_BLK, 128), lambda i: (i % 59, 0)),
            pl.BlockSpec((_P5_BLK, 8), lambda i: (i, 0)),
            pl.BlockSpec((_OUT_CH, _OUT_CH), lambda i: (0, 0)),
            pl.BlockSpec((1, _OUT_CH), lambda i: (0, 0)),
        ],
        out_specs=pl.BlockSpec((_P5_BLK, _OUT_CH), lambda i: (i, 0)),
        out_shape=jax.ShapeDtypeStruct((_NV_PAD, _OUT_CH), jnp.float32),
    )(hsum2, acc, W2, b2)


def kernel(points, full_coors, coors_inv, coors, unmask_index, batch_size,
           W1, b1, W2, b2):
    f32 = jnp.float32
    npad = _NP_PAD - _N_POINTS
    ids_pad = jnp.concatenate(
        [coors_inv, _N_VOXELS + jnp.arange(npad, dtype=jnp.int32) % (_NV_PAD - _N_VOXELS)])
    zeros8 = jnp.zeros((_NV_PAD, 8), f32)
    nb = _NP_PAD // _BLK

    pts8 = jnp.concatenate(
        [points, jnp.ones((_N_POINTS, 1), f32),
         full_coors[:, 1:].astype(f32)], axis=1)
    pts8_pad = jnp.pad(pts8, ((0, npad), (0, 0)))
    pts8p = pts8_pad.reshape(nb, 16, 128, 8).transpose(0, 2, 1, 3).reshape(
        _NP_PAD // 16, 128)
    ids_perm = ids_pad.reshape(nb, 16, 128).transpose(0, 2, 1).reshape(_NP_PAD)

    meta, acc = _run_p12(pts8p.reshape(_NP_PAD, 8), ids_perm, zeros8)
    meta128 = meta.reshape(_NP_PAD // 16, 128)

    import numpy as _np
    A = _np.zeros((8, 10), _np.float32)
    for k in range(4):
        A[k, k] = 1.0
    for k in range(3):
        A[k, 4 + k] = 1.0
        A[k, 7 + k] = 1.0
        A[5 + k, 7 + k] = -0.2
    B = _np.zeros((8, 10), _np.float32)
    for k in range(3):
        B[k, 4 + k] = -1.0
    C = _np.zeros((1, 10), _np.float32)
    C[0, 7], C[0, 8], C[0, 9] = 51.2, 51.2, 4.0
    W1a = jnp.asarray(A) @ W1
    W1b = jnp.asarray(B) @ W1
    b1p = jnp.asarray(C) @ W1 + b1[None, :]
    h2 = _run_p3(pts8p, meta128, W1a, W1b, b1p)

    hsum2 = _run_p4(h2, ids_pad, zeros8)
    features_pad = _run_p5(hsum2, acc, W2, b2[None, :])
    features = features_pad[:_N_VOXELS]

    upad = _NU_PAD - _NU
    uidx_pad = jnp.concatenate(
        [unmask_index, (jnp.arange(upad, dtype=jnp.int32) * 83) % _N_VOXELS])
    coors8 = jnp.pad(coors, ((0, 0), (0, 4)))
    pf, pc8 = _run_p6(features_pad, coors8, uidx_pad)

    partial_feature = pf[:_NU]
    partial_coors = pc8[:_NU, :4]
    voxel_features_all_one = jnp.ones((coors.shape[0], 1), f32)
    return (features, partial_feature, partial_coors, voxel_features_all_one)


# speedup vs baseline: 5.8681x; 1.0400x over previous
import functools

import jax
import jax.numpy as jnp
from jax import lax
from jax.experimental import pallas as pl
from jax.experimental.pallas import tpu as pltpu
from jax.experimental.pallas import tpu_sc as plsc

_N_POINTS = 1_000_000
_NP_PAD = 1_048_576
_NP2 = _NP_PAD // 2
_N_VOXELS = 120_000
_NV_PAD = 120_832
_NV2 = _NV_PAD // 2
_OUT_CH = 64
_NU = 60_000
_NU_PAD = 65_536

_NC, _NS = 2, 16
_BLK = 2048
_SBLK = 4096
_ROWS_PER_TILE = _NV_PAD // _NS

_mesh = functools.partial(
    plsc.VectorSubcoreMesh,
    core_axis_name="c",
    subcore_axis_name="s",
    num_cores=_NC,
    num_subcores=_NS,
)
_sc_params = functools.partial(
    pltpu.CompilerParams, use_tc_tiling_on_sc=False)


def _p12_body(pts8_hbm, idsp_hbm, zeros8_hbm, meta_hbm, acc_hbm,
              pbuf, idbuf, gbuf, acc_sh, sem):
    c = lax.axis_index("c")
    s = lax.axis_index("s")
    wid = s * _NC + c
    r0 = s * _ROWS_PER_TILE
    pltpu.sync_copy(zeros8_hbm.at[pl.ds(r0, _ROWS_PER_TILE)],
                    acc_sh.at[pl.ds(r0, _ROWS_PER_TILE)])
    plsc.subcore_barrier()
    @pl.loop(0, _NP_PAD // _NS // _SBLK)
    def _scatter(j):
        pbase = s * (_NP_PAD // _NS) + j * _SBLK
        pltpu.sync_copy(pts8_hbm.at[pl.ds(pbase, _SBLK)], pbuf)
        pltpu.sync_copy(idsp_hbm.at[pl.ds(pbase, _SBLK)], idbuf)
        pltpu.sync_copy(pbuf, acc_sh.at[idbuf], add=True)
    plsc.subcore_barrier()
    @pl.when(c == 0)
    def _():
        pltpu.sync_copy(acc_sh.at[pl.ds(r0, _ROWS_PER_TILE)],
                        acc_hbm.at[pl.ds(r0, _ROWS_PER_TILE)])
    @pl.loop(0, _NP_PAD // (_NC * _NS) // _SBLK)
    def _gather(j):
        pbase = wid * (_NP_PAD // (_NC * _NS)) + j * _SBLK
        pltpu.sync_copy(idsp_hbm.at[pl.ds(pbase, _SBLK)], idbuf)
        pltpu.async_copy(acc_sh.at[idbuf], gbuf, sem).wait()
        pltpu.sync_copy(gbuf, meta_hbm.at[pl.ds(pbase, _SBLK)])


def _run_p12(pts8_sc, ids_perm, zeros8):
    return pl.kernel(
        _p12_body,
        out_type=(
            jax.ShapeDtypeStruct((_NP_PAD, 8), jnp.float32),
            jax.ShapeDtypeStruct((_NV_PAD, 8), jnp.float32),
        ),
        mesh=_mesh(),
        compiler_params=_sc_params(),
        scratch_types=[
            pltpu.VMEM((_SBLK, 8), jnp.float32),
            pltpu.VMEM((_SBLK,), jnp.int32),
            pltpu.VMEM((_SBLK, 8), jnp.float32),
            pltpu.VMEM_SHARED((_NV_PAD, 8), jnp.float32),
            pltpu.SemaphoreType.DMA,
        ],
    )(pts8_sc, ids_perm, zeros8)


def _p4_body(h2_hbm, ids_hbm, zeros8_hbm, hsum2_hbm, hbuf, idbuf, acc_sh, sem):
    c = lax.axis_index("c")
    s = lax.axis_index("s")
    r0 = s * _ROWS_PER_TILE
    half_out = s // 8
    row0 = r0 - half_out * _NV2
    for g in range(4):
        col_g = (c * 4 + g) * 8
        pltpu.sync_copy(zeros8_hbm.at[pl.ds(r0, _ROWS_PER_TILE)],
                        acc_sh.at[pl.ds(r0, _ROWS_PER_TILE)])
        plsc.subcore_barrier()
        for hh in range(2):
            col = hh * 64 + col_g

            @pl.loop(0, _NP2 // _NS // _SBLK)
            def _scat(j):
                rbase = s * (_NP2 // _NS) + j * _SBLK
                pltpu.sync_copy(h2_hbm.at[pl.ds(rbase, _SBLK), pl.ds(col, 8)],
                                hbuf)
                pltpu.sync_copy(ids_hbm.at[pl.ds(hh * _NP2 + rbase, _SBLK)],
                                idbuf)
                pltpu.sync_copy(hbuf, acc_sh.at[idbuf], add=True)
        plsc.subcore_barrier()
        pltpu.sync_copy(
            acc_sh.at[pl.ds(r0, _ROWS_PER_TILE)],
            hsum2_hbm.at[pl.ds(row0, _ROWS_PER_TILE),
                         pl.ds(half_out * 64 + col_g, 8)])
        plsc.subcore_barrier()


def _run_p4(h2, ids_pad, zeros8):
    return pl.kernel(
        _p4_body,
        out_type=jax.ShapeDtypeStruct((_NV2, 128), jnp.float32),
        mesh=_mesh(),
        compiler_params=_sc_params(),
        scratch_types=[
            pltpu.VMEM((_SBLK, 8), jnp.float32),
            pltpu.VMEM((_SBLK,), jnp.int32),
            pltpu.VMEM_SHARED((_NV_PAD, 8), jnp.float32),
            pltpu.SemaphoreType.DMA,
        ],
    )(h2, ids_pad, zeros8)


_P6_BLK = 1024


def _p6_body(feat_hbm, coors_hbm, uidx_hbm, pf_hbm, pc_hbm,
             ubuf, fbuf, cbuf, sem):
    c = lax.axis_index("c")
    s = lax.axis_index("s")
    wid = s * _NC + c
    per_w = _NU_PAD // (_NC * _NS)

    @pl.loop(0, per_w // _P6_BLK)
    def _gat(j):
        base = wid * per_w + j * _P6_BLK
        pltpu.sync_copy(uidx_hbm.at[pl.ds(base, _P6_BLK)], ubuf)
        pltpu.async_copy(feat_hbm.at[ubuf], fbuf, sem).wait()
        pltpu.sync_copy(fbuf, pf_hbm.at[pl.ds(base, _P6_BLK)])
        pltpu.async_copy(coors_hbm.at[ubuf], cbuf, sem).wait()
        pltpu.sync_copy(cbuf, pc_hbm.at[pl.ds(base, _P6_BLK)])


def _run_p6(features, coors8, uidx_pad):
    return pl.kernel(
        _p6_body,
        out_type=(
            jax.ShapeDtypeStruct((_NU_PAD, _OUT_CH), jnp.float32),
            jax.ShapeDtypeStruct((_NU_PAD, 8), jnp.int32),
        ),
        mesh=_mesh(),
        compiler_params=_sc_params(),
        scratch_types=[
            pltpu.VMEM((_P6_BLK,), jnp.int32),
            pltpu.VMEM((_P6_BLK, _OUT_CH), jnp.float32),
            pltpu.VMEM((_P6_BLK, 8), jnp.int32),
            pltpu.SemaphoreType.DMA,
        ],
    )(features, coors8, uidx_pad)


def _unpack16(blk):
    return jnp.concatenate([blk[:, 8 * j:8 * j + 8] for j in range(16)], axis=0)


def _half_h(up8, meta8, w1a, w1b, b1p):
    mdiv = meta8 / jnp.maximum(meta8[:, 4:5], 1.0)
    z = (jnp.dot(up8, w1a, preferred_element_type=jnp.float32)
         + jnp.dot(mdiv, w1b, preferred_element_type=jnp.float32) + b1p)
    return jnp.maximum(z, 0.0)


def _p3_body(pts_lo, pts_hi, meta_lo, meta_hi, w1a_ref, w1b_ref, b1_ref,
             out_ref):
    w1a = w1a_ref[...]
    w1b = w1b_ref[...]
    b1p = b1_ref[...]
    h_lo = _half_h(_unpack16(pts_lo[...]), _unpack16(meta_lo[...]), w1a, w1b, b1p)
    h_hi = _half_h(_unpack16(pts_hi[...]), _unpack16(meta_hi[...]), w1a, w1b, b1p)
    out_ref[...] = jnp.concatenate([h_lo, h_hi], axis=1)


def _run_p3(pts8p, meta128, W1a, W1b, b1p):
    grid = _NP2 // _BLK
    nb = grid
    return pl.pallas_call(
        _p3_body,
        grid=(grid,),
        in_specs=[
            pl.BlockSpec((_BLK // 16, 128), lambda i: (i, 0)),
            pl.BlockSpec((_BLK // 16, 128), lambda i: (i + nb, 0)),
            pl.BlockSpec((_BLK // 16, 128), lambda i: (i, 0)),
            pl.BlockSpec((_BLK // 16, 128), lambda i: (i + nb, 0)),
            pl.BlockSpec((8, _OUT_CH), lambda i: (0, 0)),
            pl.BlockSpec((8, _OUT_CH), lambda i: (0, 0)),
            pl.BlockSpec((1, _OUT_CH), lambda i: (0, 0)),
        ],
        out_specs=pl.BlockSpec((_BLK, 128), lambda i: (i, 0)),
        out_shape=jax.ShapeDtypeStruct((_NP2, 128), jnp.float32),
    )(pts8p, pts8p, meta128, meta128, W1a, W1b, b1p)


_P5_BLK = 1024


def _p5_body(hsum_ref, acc_ref, w2_ref, b2_ref, out_ref):
    i = pl.program_id(0)
    half = i // (_NV2 // _P5_BLK)
    hs = hsum_ref[...]
    hsum = jnp.where(half == 0, hs[:, :64], hs[:, 64:])
    cnt = acc_ref[...][:, 4:5]
    mean = hsum / jnp.maximum(cnt, 1.0)
    feat = jnp.dot(mean, w2_ref[...], preferred_element_type=jnp.float32) + b2_ref[...]
    out_ref[...] = jnp.where(cnt > 0.0, feat, 0.0)


def _run_p5(hsum2, acc, W2, b2):
    nb = _NV2 // _P5_BLK
    grid = 2 * nb
    return pl.pallas_call(
        _p5_body,
        grid=(grid,),
        in_specs=[
            pl.BlockSpec((_P5_BLK, 128), lambda i: (i % 59, 0)),
            pl.BlockSpec((_P5_BLK, 8), lambda i: (i, 0)),
            pl.BlockSpec((_OUT_CH, _OUT_CH), lambda i: (0, 0)),
            pl.BlockSpec((1, _OUT_CH), lambda i: (0, 0)),
        ],
        out_specs=pl.BlockSpec((_P5_BLK, _OUT_CH), lambda i: (i, 0)),
        out_shape=jax.ShapeDtypeStruct((_NV_PAD, _OUT_CH), jnp.float32),
    )(hsum2, acc, W2, b2)


def kernel(points, full_coors, coors_inv, coors, unmask_index, batch_size,
           W1, b1, W2, b2):
    f32 = jnp.float32
    npad = _NP_PAD - _N_POINTS
    ids_pad = jnp.concatenate(
        [coors_inv, _N_VOXELS + jnp.arange(npad, dtype=jnp.int32) % (_NV_PAD - _N_VOXELS)])
    zeros8 = jnp.zeros((_NV_PAD, 8), f32)
    nb = _NP_PAD // _BLK

    pts8 = jnp.concatenate(
        [points, jnp.ones((_N_POINTS, 1), f32),
         full_coors[:, 1:].astype(f32)], axis=1)
    pts8_pad = jnp.pad(pts8, ((0, npad), (0, 0)))
    pts8p = pts8_pad.reshape(nb, 16, 128, 8).transpose(0, 2, 1, 3).reshape(
        _NP_PAD // 16, 128)
    ids_perm = ids_pad.reshape(nb, 16, 128).transpose(0, 2, 1).reshape(_NP_PAD)

    meta, acc = _run_p12(pts8p.reshape(_NP_PAD, 8), ids_perm, zeros8)
    meta128 = meta.reshape(_NP_PAD // 16, 128)

    import numpy as _np
    A = _np.zeros((8, 10), _np.float32)
    for k in range(4):
        A[k, k] = 1.0
    for k in range(3):
        A[k, 4 + k] = 1.0
        A[k, 7 + k] = 1.0
        A[5 + k, 7 + k] = -0.2
    B = _np.zeros((8, 10), _np.float32)
    for k in range(3):
        B[k, 4 + k] = -1.0
    C = _np.zeros((1, 10), _np.float32)
    C[0, 7], C[0, 8], C[0, 9] = 51.2, 51.2, 4.0
    W1a = jnp.asarray(A) @ W1
    W1b = jnp.asarray(B) @ W1
    b1p = jnp.asarray(C) @ W1 + b1[None, :]
    h2 = _run_p3(pts8p, meta128, W1a, W1b, b1p)

    hsum2 = _run_p4(h2, ids_pad, zeros8)
    features_pad = _run_p5(hsum2, acc, W2, b2[None, :])
    features = features_pad[:_N_VOXELS]

    upad = _NU_PAD - _NU
    uidx_pad = jnp.concatenate(
        [unmask_index, (jnp.arange(upad, dtype=jnp.int32) * 83) % _N_VOXELS])
    coors8 = jnp.pad(coors, ((0, 0), (0, 4)))
    pf, pc8 = _run_p6(features_pad, coors8, uidx_pad)

    partial_feature = pf[:_NU]
    partial_coors = pc8[:_NU, :4]
    voxel_features_all_one = jnp.ones((coors.shape[0], 1), f32)
    return (features, partial_feature, partial_coors, voxel_features_all_one)
